# Initial kernel scaffold; baseline (speedup 1.0000x reference)
import jax
import jax.numpy as jnp
from jax.experimental import pallas as pl


def kernel(input, table):
    raise NotImplementedError("write your pallas kernel here")


# speedup vs baseline: 1.1027x; 1.1027x over previous
import functools

import jax
import jax.numpy as jnp
from jax import lax
from jax.experimental import pallas as pl
from jax.experimental.pallas import tpu as pltpu
from jax.experimental.pallas import tpu_sc as plsc

_B_TOTAL = 16384 * 50
_D = 32
_NW = 32
_B_PER_W = _B_TOTAL // _NW
_CHUNK = 1600
_NCHUNK = _B_PER_W // _CHUNK


@functools.partial(
    pl.kernel,
    mesh=plsc.VectorSubcoreMesh(core_axis_name="c", subcore_axis_name="s"),
    out_type=jax.ShapeDtypeStruct((_B_TOTAL, _D), jnp.float32),
    scratch_types=[
        pltpu.VMEM((_CHUNK,), jnp.int32),
        pltpu.VMEM((_CHUNK, _D), jnp.float32),
        pltpu.SemaphoreType.DMA,
    ],
    compiler_params=pltpu.CompilerParams(use_tc_tiling_on_sc=False),
)
def _emb_lookup(idx_hbm, table_hbm, out_hbm, idx_v, rows_v, sem):
    wid = lax.axis_index("s") * 2 + lax.axis_index("c")
    base = wid * _B_PER_W

    def body(i, carry):
        off = base + i * _CHUNK
        pltpu.sync_copy(idx_hbm.at[pl.ds(off, _CHUNK)], idx_v)
        pltpu.async_copy(table_hbm.at[idx_v], rows_v, sem).wait()
        pltpu.sync_copy(rows_v, out_hbm.at[pl.ds(off, _CHUNK)])
        return carry

    lax.fori_loop(0, _NCHUNK, body, 0)


def kernel(input, table):
    flat_idx = input.reshape(-1)
    out = _emb_lookup(flat_idx, table)
    return out.reshape(input.shape + (table.shape[1],))


# speedup vs baseline: 1.1100x; 1.0066x over previous
import functools

import jax
import jax.numpy as jnp
from jax import lax
from jax.experimental import pallas as pl
from jax.experimental.pallas import tpu as pltpu
from jax.experimental.pallas import tpu_sc as plsc

_B_TOTAL = 16384 * 50
_D = 32
_NW = 32
_B_PER_W = _B_TOTAL // _NW
_CHUNK = 1600
_NCHUNK = _B_PER_W // _CHUNK
_NBUF = 2


@functools.partial(
    pl.kernel,
    mesh=plsc.VectorSubcoreMesh(core_axis_name="c", subcore_axis_name="s"),
    out_type=jax.ShapeDtypeStruct((_B_TOTAL, _D), jnp.float32),
    scratch_types=[
        pltpu.VMEM((_CHUNK,), jnp.int32),
        pltpu.VMEM((_CHUNK,), jnp.int32),
        pltpu.VMEM((_CHUNK, _D), jnp.float32),
        pltpu.VMEM((_CHUNK, _D), jnp.float32),
        pltpu.SemaphoreType.DMA,
        pltpu.SemaphoreType.DMA,
        pltpu.SemaphoreType.DMA,
        pltpu.SemaphoreType.DMA,
        pltpu.SemaphoreType.DMA,
        pltpu.SemaphoreType.DMA,
    ],
    compiler_params=pltpu.CompilerParams(use_tc_tiling_on_sc=False),
)
def _emb_lookup(idx_hbm, table_hbm, out_hbm, idx_v0, idx_v1, rows_v0, rows_v1,
                is0, is1, gs0, gs1, os0, os1):
    idx_bufs = [idx_v0, idx_v1]
    rows_bufs = [rows_v0, rows_v1]
    idx_sems = [is0, is1]
    gat_sems = [gs0, gs1]
    out_sems = [os0, os1]
    wid = lax.axis_index("s") * 2 + lax.axis_index("c")
    base = wid * _B_PER_W

    for b in range(_NBUF):
        pltpu.async_copy(
            idx_hbm.at[pl.ds(base + b * _CHUNK, _CHUNK)], idx_bufs[b],
            idx_sems[b])

    def outer(o, carry):
        for b in range(_NBUF):
            g = o * _NBUF + b
            off = base + g * _CHUNK

            pltpu.make_async_copy(
                idx_hbm.at[pl.ds(off, _CHUNK)], idx_bufs[b],
                idx_sems[b]).wait()

            @pl.when(o > 0)
            def _wait_store():
                pltpu.make_async_copy(
                    rows_bufs[b], out_hbm.at[pl.ds(off, _CHUNK)],
                    out_sems[b]).wait()

            pltpu.async_copy(
                table_hbm.at[idx_bufs[b]], rows_bufs[b], gat_sems[b]).wait()

            @pl.when(g + _NBUF < _NCHUNK)
            def _prefetch_idx():
                pltpu.async_copy(
                    idx_hbm.at[pl.ds(off + _NBUF * _CHUNK, _CHUNK)],
                    idx_bufs[b], idx_sems[b])

            pltpu.async_copy(
                rows_bufs[b], out_hbm.at[pl.ds(off, _CHUNK)], out_sems[b])
        return carry

    lax.fori_loop(0, _NCHUNK // _NBUF, outer, 0)

    for b in range(_NBUF):
        pltpu.make_async_copy(
            rows_bufs[b], out_hbm.at[pl.ds(base, _CHUNK)],
            out_sems[b]).wait()


def kernel(input, table):
    flat_idx = input.reshape(-1)
    out = _emb_lookup(flat_idx, table)
    return out.reshape(input.shape + (table.shape[1],))


# speedup vs baseline: 1.1139x; 1.0035x over previous
import functools

import jax
import jax.numpy as jnp
from jax import lax
from jax.experimental import pallas as pl
from jax.experimental.pallas import tpu as pltpu
from jax.experimental.pallas import tpu_sc as plsc

_B_TOTAL = 16384 * 50
_D = 32
_NW = 32
_B_PER_W = _B_TOTAL // _NW
_CHUNK = 800
_NCHUNK = _B_PER_W // _CHUNK
_NBUF = 4


@functools.partial(
    pl.kernel,
    mesh=plsc.VectorSubcoreMesh(core_axis_name="c", subcore_axis_name="s"),
    out_type=jax.ShapeDtypeStruct((_B_TOTAL, _D), jnp.float32),
    scratch_types=(
        [pltpu.VMEM((_CHUNK,), jnp.int32) for _ in range(_NBUF)]
        + [pltpu.VMEM((_CHUNK, _D), jnp.float32) for _ in range(_NBUF)]
        + [pltpu.SemaphoreType.DMA for _ in range(3 * _NBUF)]
    ),
    compiler_params=pltpu.CompilerParams(use_tc_tiling_on_sc=False),
)
def _emb_lookup(idx_hbm, table_hbm, out_hbm, *scratch):
    idx_bufs = list(scratch[:_NBUF])
    rows_bufs = list(scratch[_NBUF:2 * _NBUF])
    idx_sems = list(scratch[2 * _NBUF:3 * _NBUF])
    gat_sems = list(scratch[3 * _NBUF:4 * _NBUF])
    out_sems = list(scratch[4 * _NBUF:5 * _NBUF])
    wid = lax.axis_index("s") * 2 + lax.axis_index("c")
    base = wid * _B_PER_W

    for b in range(_NBUF):
        pltpu.async_copy(
            idx_hbm.at[pl.ds(base + b * _CHUNK, _CHUNK)], idx_bufs[b],
            idx_sems[b])

    def step(g, b, pb, first):
        off = base + g * _CHUNK

        pltpu.make_async_copy(
            idx_hbm.at[pl.ds(off, _CHUNK)], idx_bufs[b], idx_sems[b]).wait()

        def _wait_store():
            pltpu.make_async_copy(
                rows_bufs[b], out_hbm.at[pl.ds(off, _CHUNK)],
                out_sems[b]).wait()

        if isinstance(g, int):
            if g >= _NBUF:
                _wait_store()
        else:
            pl.when(g >= _NBUF)(_wait_store)

        pltpu.async_copy(table_hbm.at[idx_bufs[b]], rows_bufs[b], gat_sems[b])

        if not first:
            poff = base + (g - 1) * _CHUNK
            pltpu.make_async_copy(
                table_hbm.at[idx_bufs[pb]], rows_bufs[pb],
                gat_sems[pb]).wait()

            def _prefetch_idx():
                pltpu.async_copy(
                    idx_hbm.at[pl.ds(poff + _NBUF * _CHUNK, _CHUNK)],
                    idx_bufs[pb], idx_sems[pb])

            if isinstance(g, int):
                if g - 1 + _NBUF < _NCHUNK:
                    _prefetch_idx()
            else:
                pl.when(g - 1 + _NBUF < _NCHUNK)(_prefetch_idx)

            pltpu.async_copy(
                rows_bufs[pb], out_hbm.at[pl.ds(poff, _CHUNK)], out_sems[pb])

    step(0, 0, _NBUF - 1, True)

    def outer(o, carry):
        for b in range(_NBUF):
            g = o * _NBUF + b + 1
            step(g, (b + 1) % _NBUF, b, False)
        return carry

    lax.fori_loop(0, (_NCHUNK - 1) // _NBUF, outer, 0)

    for g in range(((_NCHUNK - 1) // _NBUF) * _NBUF + 1, _NCHUNK):
        step(g, g % _NBUF, (g - 1) % _NBUF, False)

    lb = (_NCHUNK - 1) % _NBUF
    loff = base + (_NCHUNK - 1) * _CHUNK
    pltpu.make_async_copy(
        table_hbm.at[idx_bufs[lb]], rows_bufs[lb], gat_sems[lb]).wait()
    pltpu.async_copy(
        rows_bufs[lb], out_hbm.at[pl.ds(loff, _CHUNK)], out_sems[lb])

    for b in range(_NBUF):
        pltpu.make_async_copy(
            rows_bufs[b], out_hbm.at[pl.ds(base, _CHUNK)],
            out_sems[b]).wait()


def kernel(input, table):
    flat_idx = input.reshape(-1)
    out = _emb_lookup(flat_idx, table)
    return out.reshape(input.shape + (table.shape[1],))
